# speedup vs baseline: 6.6292x; 6.6292x over previous
import functools

import jax
import jax.numpy as jnp
from jax import lax
from jax.experimental import pallas as pl
from jax.experimental.pallas import tpu as pltpu
from jax.experimental.pallas import tpu_sc as plsc

EMBED_DIM = 128
CHUNK = 128
NBUF = 4


@functools.lru_cache(maxsize=None)
def _make_gather(B: int, vocab: int, d: int):
    info = plsc.get_sparse_core_info()
    nc, ns = info.num_cores, info.num_subcores
    nw = nc * ns
    assert B % (nw * CHUNK) == 0
    b_per_w = B // nw
    n_ch = b_per_w // CHUNK
    assert n_ch % NBUF == 0
    t_outer = n_ch // NBUF

    mesh = plsc.VectorSubcoreMesh(core_axis_name="c", subcore_axis_name="s")

    @functools.partial(
        pl.kernel,
        mesh=mesh,
        out_type=jax.ShapeDtypeStruct((B, d), jnp.float32),
        scratch_types=[
            pltpu.VMEM((NBUF, CHUNK), jnp.int32),
            pltpu.VMEM((NBUF, CHUNK, d), jnp.float32),
        ]
        + [pltpu.SemaphoreType.DMA] * (2 * NBUF),
    )
    def k(ids_hbm, table_hbm, out_hbm, idx_v, rows_v, *sems):
        gsem = sems[:NBUF]
        osem = sems[NBUF:]
        wid = lax.axis_index("s") * nc + lax.axis_index("c")
        base = wid * b_per_w

        def fetch(g, b):
            off = base + g * CHUNK
            pltpu.sync_copy(ids_hbm.at[pl.ds(off, CHUNK)], idx_v.at[b])
            pltpu.async_copy(table_hbm.at[idx_v.at[b]], rows_v.at[b], gsem[b])

        def wait_gather(b):
            pltpu.make_async_copy(
                table_hbm.at[idx_v.at[b]], rows_v.at[b], gsem[b]
            ).wait()

        def out_start(g, b):
            off = base + g * CHUNK
            pltpu.async_copy(rows_v.at[b], out_hbm.at[pl.ds(off, CHUNK)], osem[b])

        def wait_out(g, b):
            off = base + g * CHUNK
            pltpu.make_async_copy(
                rows_v.at[b], out_hbm.at[pl.ds(off, CHUNK)], osem[b]
            ).wait()

        for b in range(NBUF):
            fetch(b, b)

        def body(t, carry):
            for b in range(NBUF):
                g = t * NBUF + b
                wait_gather(b)
                out_start(g, b)
            for b in range(NBUF):
                g = t * NBUF + b

                @pl.when(t + 1 < t_outer)
                def _():
                    wait_out(g, b)
                    fetch(g + NBUF, b)

            return carry

        lax.fori_loop(0, t_outer, body, 0)

        for b in range(NBUF):
            g = (t_outer - 1) * NBUF + b
            wait_out(g, b)

    return k


def kernel(input_ids, weight):
    bsz, seq = input_ids.shape
    vocab, d = weight.shape
    ids = input_ids.reshape(-1).astype(jnp.int32)
    out = _make_gather(bsz * seq, vocab, d)(ids, weight)
    return out.reshape(bsz, seq, d)
